# Initial kernel scaffold; baseline (speedup 1.0000x reference)
import jax
import jax.numpy as jnp
from jax.experimental import pallas as pl


def kernel(x, edge_index, root_mask, W_embed, b_embed, W_layers, b_layers, ln_g, ln_b, W_out, b_out):
    raise NotImplementedError("write your pallas kernel here")


# speedup vs baseline: 4.9860x; 4.9860x over previous
import functools

import jax
import jax.numpy as jnp
from jax import lax
from jax.experimental import pallas as pl
from jax.experimental.pallas import tpu as pltpu
from jax.experimental.pallas import tpu_sc as plsc

_N = 10000
_E = 320000
_D = 128
_OUT = 7
_DEPTH = 3

_NC = 2
_NS = 16
_NW = _NC * _NS
_CHUNK = 128
_CPW = 80
_EPAD = _NW * _CPW * _CHUNK
_NPAD = 10240
_RPT = _NPAD // _NS
_DEGW = 16

_GRP = 16

_mesh = plsc.VectorSubcoreMesh(core_axis_name="c", subcore_axis_name="s")


def _agg_body(g_hbm, src_hbm, dst_hbm, z_hbm, out_hbm,
              src_v, dst_v, bufa, bufb, acc, sema, semb):
    cid = lax.axis_index("c")
    sid = lax.axis_index("s")
    wid = sid * _NC + cid

    pltpu.sync_copy(z_hbm, bufa)

    def _zero(j, c):
        pltpu.sync_copy(bufa, acc.at[pl.ds(sid * _RPT + j * _CHUNK, _CHUNK)])
        return c

    lax.fori_loop(0, _RPT // _CHUNK, _zero, 0)
    plsc.subcore_barrier()

    def _group(gidx, c):
        base = wid * _CPW + gidx * _GRP
        pltpu.sync_copy(src_hbm.at[pl.ds(base, _GRP)], src_v)
        pltpu.sync_copy(dst_hbm.at[pl.ds(base, _GRP)], dst_v)
        pltpu.async_copy(g_hbm.at[src_v.at[0]], bufa, sema)

        def _pair(t, c2):
            k0 = 2 * t
            pltpu.async_copy(g_hbm.at[src_v.at[k0 + 1]], bufb, semb)
            pltpu.make_async_copy(g_hbm.at[src_v.at[k0]], bufa, sema).wait()
            pltpu.sync_copy(bufa, acc.at[dst_v.at[k0]], add=True)
            kn = jnp.minimum(k0 + 2, _GRP - 1)
            pltpu.async_copy(g_hbm.at[src_v.at[kn]], bufa, sema)
            pltpu.make_async_copy(g_hbm.at[src_v.at[k0 + 1]], bufb, semb).wait()
            pltpu.sync_copy(bufb, acc.at[dst_v.at[k0 + 1]], add=True)
            return c2

        lax.fori_loop(0, _GRP // 2, _pair, 0)
        pltpu.make_async_copy(g_hbm.at[src_v.at[_GRP - 1]], bufa, sema).wait()
        return c

    lax.fori_loop(0, _CPW // _GRP, _group, 0)
    plsc.subcore_barrier()

    def _dump(j, c):
        r = sid * _RPT + j * _CHUNK
        pltpu.sync_copy(acc.at[pl.ds(r, _CHUNK)], out_hbm.at[cid, pl.ds(r, _CHUNK)])
        return c

    lax.fori_loop(0, _RPT // _CHUNK, _dump, 0)


_agg_call = functools.partial(
    pl.kernel,
    mesh=_mesh,
    out_type=jax.ShapeDtypeStruct((_NC, _NPAD, _D), jnp.float32),
    scratch_types=[
        pltpu.VMEM((_GRP, _CHUNK), jnp.int32),
        pltpu.VMEM((_GRP, _CHUNK), jnp.int32),
        pltpu.VMEM((_CHUNK, _D), jnp.float32),
        pltpu.VMEM((_CHUNK, _D), jnp.float32),
        pltpu.VMEM_SHARED((_NPAD, _D), jnp.float32),
        pltpu.SemaphoreType.DMA,
        pltpu.SemaphoreType.DMA,
    ],
)(_agg_body)


_BR = 1024
_GRID = _NPAD // _BR

_row_spec = pl.BlockSpec((_BR, _D), lambda i: (i, 0))
_mat_spec = pl.BlockSpec((_D, _D), lambda i: (0, 0))
_vec_spec = pl.BlockSpec((1, _D), lambda i: (0, 0))


def _rs_body(d0_ref, d1_ref, rs_ref):
    d = d0_ref[...] + d1_ref[...]
    rs_ref[...] = lax.rsqrt(jnp.maximum(d, 1.0))


_rs_call = pl.pallas_call(
    _rs_body,
    out_shape=jax.ShapeDtypeStruct((_NPAD // _D, _D), jnp.float32),
)


def _embed_body(x_ref, w_ref, b_ref, rs_ref, h_ref, g_ref):
    h = jnp.dot(x_ref[...], w_ref[...], preferred_element_type=jnp.float32)
    h = h + b_ref[...]
    h_ref[...] = h
    g_ref[...] = h * rs_ref[...]


_embed_call = pl.pallas_call(
    _embed_body,
    grid=(_GRID,),
    in_specs=[_row_spec, _mat_spec, _vec_spec, _row_spec],
    out_specs=[_row_spec, _row_spec],
    out_shape=[
        jax.ShapeDtypeStruct((_NPAD, _D), jnp.float32),
        jax.ShapeDtypeStruct((_NPAD, _D), jnp.float32),
    ],
)


def _layer_math(p0_ref, p1_ref, rs_ref, h_ref, w_ref, b_ref, lg_ref, lb_ref):
    agg = (p0_ref[...] + p1_ref[...]) * rs_ref[...]
    t = jnp.dot(agg, w_ref[...], preferred_element_type=jnp.float32)
    t = jnp.maximum(t + b_ref[...], 0.0)
    u = h_ref[...] + t
    mu = jnp.mean(u, axis=-1, keepdims=True)
    dlt = u - mu
    var = jnp.mean(dlt * dlt, axis=-1, keepdims=True)
    return dlt * lax.rsqrt(var + 1e-5) * lg_ref[...] + lb_ref[...]


def _upd_body(p0_ref, p1_ref, rs_ref, h_ref, w_ref, b_ref, lg_ref, lb_ref,
              hn_ref, gn_ref):
    nh = _layer_math(p0_ref, p1_ref, rs_ref, h_ref, w_ref, b_ref, lg_ref, lb_ref)
    hn_ref[...] = nh
    gn_ref[...] = nh * rs_ref[...]


_upd_call = pl.pallas_call(
    _upd_body,
    grid=(_GRID,),
    in_specs=[_row_spec, _row_spec, _row_spec, _row_spec,
              _mat_spec, _vec_spec, _vec_spec, _vec_spec],
    out_specs=[_row_spec, _row_spec],
    out_shape=[
        jax.ShapeDtypeStruct((_NPAD, _D), jnp.float32),
        jax.ShapeDtypeStruct((_NPAD, _D), jnp.float32),
    ],
)


def _fin_body(p0_ref, p1_ref, rs_ref, h_ref, w_ref, b_ref, lg_ref, lb_ref,
              wo_ref, bo_ref, mk_ref, o_ref):
    nh = _layer_math(p0_ref, p1_ref, rs_ref, h_ref, w_ref, b_ref, lg_ref, lb_ref)
    o = jnp.dot(nh, wo_ref[...], preferred_element_type=jnp.float32) + bo_ref[...]
    o_ref[...] = jnp.where(mk_ref[...] > 0.0, o, 0.0)


_fin_call = pl.pallas_call(
    _fin_body,
    grid=(_GRID,),
    in_specs=[_row_spec, _row_spec, _row_spec, _row_spec,
              _mat_spec, _vec_spec, _vec_spec, _vec_spec,
              _mat_spec, _vec_spec, _row_spec],
    out_specs=pl.BlockSpec((_BR, _D), lambda i: (i, 0)),
    out_shape=jax.ShapeDtypeStruct((_NPAD, _D), jnp.float32),
)


def kernel(x, edge_index, root_mask, W_embed, b_embed, W_layers, b_layers,
           ln_g, ln_b, W_out, b_out):
    src = edge_index[0]
    dst = edge_index[1]
    pad_e = _EPAD - _E
    src_p = jnp.concatenate(
        [src, jnp.zeros((pad_e,), jnp.int32)]).reshape(_EPAD // _CHUNK, _CHUNK)
    dst_p = jnp.concatenate(
        [dst, jnp.full((pad_e,), _N, jnp.int32)]).reshape(_EPAD // _CHUNK, _CHUNK)
    x_p = jnp.concatenate([x, jnp.zeros((_NPAD - _N, _D), jnp.float32)], axis=0)
    zblk = jnp.zeros((_CHUNK, _D), jnp.float32)

    ones_g = jnp.ones((_NPAD, _D), jnp.float32)
    degp = _agg_call(ones_g, src_p, dst_p, zblk)
    deg0 = degp[0, :, 0].reshape(_NPAD // _D, _D)
    deg1 = degp[1, :, 0].reshape(_NPAD // _D, _D)
    rs = _rs_call(deg0, deg1)
    rs_b = jnp.broadcast_to(rs.reshape(_NPAD, 1), (_NPAD, _D))

    h, g = _embed_call(x_p, W_embed, b_embed.reshape(1, _D), rs_b)

    for i in range(_DEPTH - 1):
        p = _agg_call(g, src_p, dst_p, zblk)
        h, g = _upd_call(p[0], p[1], rs_b, h, W_layers[i],
                         b_layers[i].reshape(1, _D), ln_g[i].reshape(1, _D),
                         ln_b[i].reshape(1, _D))

    p = _agg_call(g, src_p, dst_p, zblk)
    wo = jnp.zeros((_D, _D), jnp.float32).at[:, :_OUT].set(W_out)
    bo = jnp.zeros((1, _D), jnp.float32).at[0, :_OUT].set(b_out)
    mk = jnp.concatenate([root_mask.astype(jnp.float32),
                          jnp.zeros((_NPAD - _N,), jnp.float32)])
    mk_b = jnp.broadcast_to(mk.reshape(_NPAD, 1), (_NPAD, _D))
    out = _fin_call(p[0], p[1], rs_b, h, W_layers[_DEPTH - 1],
                    b_layers[_DEPTH - 1].reshape(1, _D),
                    ln_g[_DEPTH - 1].reshape(1, _D),
                    ln_b[_DEPTH - 1].reshape(1, _D), wo, bo, mk_b)
    return out[:_N, :_OUT]
